# baseline (speedup 1.0000x reference)
import jax
import jax.numpy as jnp
from jax.experimental import pallas as pl

B = 16384
D = 128
TB = 8


def _dot_body(gu_ref, gi_ref, out_ref):
    prod = gu_ref[...] * gi_ref[...]
    pt = jnp.swapaxes(prod, 1, 2)
    out_ref[...] = jnp.sum(pt, axis=1)


@jax.jit
def kernel(gu, gi):
    gu3 = gu.reshape(B // D, D, D)
    gi3 = gi.reshape(B // D, D, D)
    out = pl.pallas_call(
        _dot_body,
        grid=(B // D // TB,),
        in_specs=[
            pl.BlockSpec((TB, D, D), lambda i: (i, 0, 0)),
            pl.BlockSpec((TB, D, D), lambda i: (i, 0, 0)),
        ],
        out_specs=pl.BlockSpec((TB, D), lambda i: (i, 0)),
        out_shape=jax.ShapeDtypeStruct((B // D, D), jnp.float32),
    )(gu3, gi3)
    return out.reshape(B)


# speedup vs baseline: 1.7332x; 1.7332x over previous
import jax
import jax.numpy as jnp
from jax.experimental import pallas as pl

B = 16384
D = 128
TB = 32


def _dot_body(gu_ref, gi_ref, out_ref):
    prod = gu_ref[...] * gi_ref[...]
    pt = jnp.swapaxes(prod, 1, 2)
    out_ref[...] = jnp.sum(pt, axis=1)


@jax.jit
def kernel(gu, gi):
    gu3 = gu.reshape(B // D, D, D)
    gi3 = gi.reshape(B // D, D, D)
    out = pl.pallas_call(
        _dot_body,
        grid=(B // D // TB,),
        in_specs=[
            pl.BlockSpec((TB, D, D), lambda i: (i, 0, 0)),
            pl.BlockSpec((TB, D, D), lambda i: (i, 0, 0)),
        ],
        out_specs=pl.BlockSpec((TB, D), lambda i: (i, 0)),
        out_shape=jax.ShapeDtypeStruct((B // D, D), jnp.float32),
    )(gu3, gi3)
    return out.reshape(B)


# speedup vs baseline: 1.7390x; 1.0033x over previous
import jax
import jax.numpy as jnp
from jax.experimental import pallas as pl

B = 16384
D = 128
TB = 64


def _dot_body(gu_ref, gi_ref, out_ref):
    prod = gu_ref[...] * gi_ref[...]
    pt = jnp.swapaxes(prod, 1, 2)
    out_ref[...] = jnp.sum(pt, axis=1)


@jax.jit
def kernel(gu, gi):
    gu3 = gu.reshape(B // D, D, D)
    gi3 = gi.reshape(B // D, D, D)
    out = pl.pallas_call(
        _dot_body,
        grid=(B // D // TB,),
        in_specs=[
            pl.BlockSpec((TB, D, D), lambda i: (i, 0, 0)),
            pl.BlockSpec((TB, D, D), lambda i: (i, 0, 0)),
        ],
        out_specs=pl.BlockSpec((TB, D), lambda i: (i, 0)),
        out_shape=jax.ShapeDtypeStruct((B // D, D), jnp.float32),
    )(gu3, gi3)
    return out.reshape(B)


# speedup vs baseline: 1.8294x; 1.0520x over previous
import jax
import jax.numpy as jnp
from jax.experimental import pallas as pl
from jax.experimental.pallas import tpu as pltpu

B = 16384
D = 128
NT = B // D
CT = 16
NCH = NT // CT
K = 3


def _dot_body(gu_hbm, gi_hbm, out_hbm, gu_v, gi_v, out_v, sem_u, sem_i,
              sem_o):
    def copies(c):
        k = c % K
        cu = pltpu.make_async_copy(
            gu_hbm.at[pl.ds(c * CT, CT)], gu_v.at[k], sem_u.at[k])
        ci = pltpu.make_async_copy(
            gi_hbm.at[pl.ds(c * CT, CT)], gi_v.at[k], sem_i.at[k])
        return cu, ci

    def start(c):
        cu, ci = copies(c)
        cu.start()
        ci.start()

    for c in range(K - 1):
        start(c)
    for c in range(NCH):
        if c + K - 1 < NCH:
            start(c + K - 1)
        cu, ci = copies(c)
        cu.wait()
        ci.wait()
        k = c % K
        prod = gu_v[k] * gi_v[k]
        pt = jnp.swapaxes(prod, 1, 2)
        out_v[pl.ds(c * CT, CT), :] = jnp.sum(pt, axis=1)
    co = pltpu.make_async_copy(out_v, out_hbm, sem_o)
    co.start()
    co.wait()


@jax.jit
def kernel(gu, gi):
    gu3 = gu.reshape(NT, D, D)
    gi3 = gi.reshape(NT, D, D)
    out = pl.pallas_call(
        _dot_body,
        in_specs=[
            pl.BlockSpec(memory_space=pl.ANY),
            pl.BlockSpec(memory_space=pl.ANY),
        ],
        out_specs=pl.BlockSpec(memory_space=pl.ANY),
        out_shape=jax.ShapeDtypeStruct((NT, D), jnp.float32),
        scratch_shapes=[
            pltpu.VMEM((K, CT, D, D), jnp.float32),
            pltpu.VMEM((K, CT, D, D), jnp.float32),
            pltpu.VMEM((NT, D), jnp.float32),
            pltpu.SemaphoreType.DMA((K,)),
            pltpu.SemaphoreType.DMA((K,)),
            pltpu.SemaphoreType.DMA,
        ],
    )(gu3, gi3)
    return out.reshape(B)


# speedup vs baseline: 1.9399x; 1.0604x over previous
import jax
import jax.numpy as jnp
from jax.experimental import pallas as pl
from jax.experimental.pallas import tpu as pltpu

B = 16384
D = 128
NT = B // D
CT = 16
NCH = NT // CT
K = 4


def _dot_body(gu_hbm, gi_hbm, out_hbm, gu_v, gi_v, out_v, sem_u, sem_i,
              sem_o):
    def copies(c):
        k = c % K
        cu = pltpu.make_async_copy(
            gu_hbm.at[pl.ds(c * CT, CT)], gu_v.at[k], sem_u.at[k])
        ci = pltpu.make_async_copy(
            gi_hbm.at[pl.ds(c * CT, CT)], gi_v.at[k], sem_i.at[k])
        return cu, ci

    def start(c):
        cu, ci = copies(c)
        cu.start()
        ci.start()

    for c in range(K - 1):
        start(c)
    for c in range(NCH):
        if c + K - 1 < NCH:
            start(c + K - 1)
        cu, ci = copies(c)
        cu.wait()
        ci.wait()
        k = c % K
        prod = gu_v[k] * gi_v[k]
        pt = jnp.swapaxes(prod, 1, 2)
        out_v[pl.ds(c * CT, CT), :] = jnp.sum(pt, axis=1)
    co = pltpu.make_async_copy(out_v, out_hbm, sem_o)
    co.start()
    co.wait()


@jax.jit
def kernel(gu, gi):
    gu3 = gu.reshape(NT, D, D)
    gi3 = gi.reshape(NT, D, D)
    out = pl.pallas_call(
        _dot_body,
        in_specs=[
            pl.BlockSpec(memory_space=pl.ANY),
            pl.BlockSpec(memory_space=pl.ANY),
        ],
        out_specs=pl.BlockSpec(memory_space=pl.ANY),
        out_shape=jax.ShapeDtypeStruct((NT, D), jnp.float32),
        scratch_shapes=[
            pltpu.VMEM((K, CT, D, D), jnp.float32),
            pltpu.VMEM((K, CT, D, D), jnp.float32),
            pltpu.VMEM((NT, D), jnp.float32),
            pltpu.SemaphoreType.DMA((K,)),
            pltpu.SemaphoreType.DMA((K,)),
            pltpu.SemaphoreType.DMA,
        ],
    )(gu3, gi3)
    return out.reshape(B)
